# baseline (speedup 1.0000x reference)
import functools

import jax
import jax.numpy as jnp
from jax import lax
from jax.experimental import pallas as pl
from jax.experimental.pallas import tpu as pltpu
from jax.experimental.pallas import tpu_sc as plsc

NC = 2
NS = 16
NW = NC * NS

SA = 16
ST = 4
PSTRIDE = 64


@functools.lru_cache(maxsize=None)
def _make_emb(N: int, S: int, D: int):
    npw = N // NW
    assert npw % 2 == 0
    npairs = npw // 2
    mesh = plsc.VectorSubcoreMesh(core_axis_name="c", subcore_axis_name="s")

    @functools.partial(
        pl.kernel,
        mesh=mesh,
        out_type=(
            jax.ShapeDtypeStruct((N * SA, D), jnp.float32),
            jax.ShapeDtypeStruct((N * ST, D), jnp.float32),
        ),
        scratch_types=[
            pltpu.VMEM((npairs * PSTRIDE,), jnp.int32),
            pltpu.VMEM((SA, D), jnp.float32),
            pltpu.VMEM((SA, D), jnp.float32),
            pltpu.VMEM((2 * ST, D), jnp.float32),
            pltpu.VMEM((2 * ST, D), jnp.float32),
            pltpu.SemaphoreType.DMA,
            pltpu.SemaphoreType.DMA,
            pltpu.SemaphoreType.DMA,
            pltpu.SemaphoreType.DMA,
            pltpu.SemaphoreType.DMA,
            pltpu.SemaphoreType.DMA,
            pltpu.SemaphoreType.DMA,
            pltpu.SemaphoreType.DMA,
        ],
    )
    def emb(table_hbm, idx_hbm, out_hbm, tails_hbm, idx_v,
            mb0, mb1, tb0, tb1, g0, g1, s0, s1, tg0, tg1, w0, w1):
        wid = lax.axis_index("s") * NC + lax.axis_index("c")
        base = wid * npw
        mbufs, gsem, ssem = (mb0, mb1), (g0, g1), (s0, s1)
        tbufs, tgsem, wsem = (tb0, tb1), (tg0, tg1), (w0, w1)
        pltpu.sync_copy(
            idx_hbm.at[pl.ds(wid * npairs * PSTRIDE, npairs * PSTRIDE)],
            idx_v)

        def midx(c):
            return idx_v.at[pl.ds((c // 2) * PSTRIDE + (c % 2) * SA, SA)]

        def tidx(p):
            return idx_v.at[pl.ds(p * PSTRIDE + 2 * SA, 2 * ST)]

        def mg_start(c, b):
            pltpu.async_copy(table_hbm.at[midx(c)], mbufs[b], gsem[b])

        def mg_wait(c, b):
            pltpu.make_async_copy(
                table_hbm.at[midx(c)], mbufs[b], gsem[b]).wait()

        def ms_slice(c):
            return out_hbm.at[pl.ds((base + c) * SA, SA)]

        def tg_start(p, t):
            pltpu.async_copy(table_hbm.at[tidx(p)], tbufs[t], tgsem[t])

        def tg_wait(p, t):
            pltpu.make_async_copy(
                table_hbm.at[tidx(p)], tbufs[t], tgsem[t]).wait()

        def ts_slice(p):
            return tails_hbm.at[pl.ds((base + 2 * p) * ST, 2 * ST)]

        mg_start(0, 0)
        mg_start(1, 1)
        tg_start(0, 0)
        tg_start(1, 1)

        def body(p, carry):
            for h in range(2):
                c = 2 * p + h
                b = h
                mg_wait(c, b)
                pltpu.async_copy(mbufs[b], ms_slice(c), ssem[b])
                pltpu.make_async_copy(mbufs[b], ms_slice(c), ssem[b]).wait()

                @pl.when(p < npairs - 1)
                def _():
                    mg_start(c + 2, b)

            t = lax.rem(p, 2)
            for tt in range(2):

                @pl.when(t == tt)
                def _():
                    tg_wait(p, tt)
                    pltpu.async_copy(tbufs[tt], ts_slice(p), wsem[tt])
                    pltpu.make_async_copy(
                        tbufs[tt], ts_slice(p), wsem[tt]).wait()

                    @pl.when(p < npairs - 2)
                    def _():
                        tg_start(p + 2, tt)

            return carry

        lax.fori_loop(0, npairs, body, 0)

    return emb


def _tc_merge_body(main_ref, tails_ref, out_ref):
    for k in range(out_ref.shape[0]):
        out_ref[k, pl.ds(0, SA)] = main_ref[pl.ds(k * SA, SA)]
        out_ref[k, pl.ds(SA, ST)] = tails_ref[pl.ds(k * ST, ST)]


@functools.lru_cache(maxsize=None)
def _make_merge(N: int, S: int, D: int):
    BI = 16
    return pl.pallas_call(
        _tc_merge_body,
        grid=(N // BI,),
        in_specs=[
            pl.BlockSpec((BI * SA, D), lambda i: (i, 0)),
            pl.BlockSpec((BI * ST, D), lambda i: (i, 0)),
        ],
        out_specs=pl.BlockSpec((BI, S, D), lambda i: (i, 0, 0)),
        out_shape=jax.ShapeDtypeStruct((N, S, D), jnp.float32),
    )


def kernel(x, table):
    N, S = x.shape
    D = table.shape[1]
    npw = N // NW
    x4 = x.astype(jnp.int32).reshape(NW, npw // 2, 2, S)
    main = x4[..., :SA].reshape(NW, npw // 2, 2 * SA)
    tails_idx = x4[..., SA:].reshape(NW, npw // 2, 2 * ST)
    pad = jnp.zeros((NW, npw // 2, PSTRIDE - 2 * SA - 2 * ST), jnp.int32)
    xp = jnp.concatenate([main, tails_idx, pad], axis=-1).reshape(-1)
    main_rows, tail_rows = _make_emb(N, S, D)(table, xp)
    return _make_merge(N, S, D)(main_rows, tail_rows)


# speedup vs baseline: 1.3349x; 1.3349x over previous
import functools

import jax
import jax.numpy as jnp
from jax import lax
from jax.experimental import pallas as pl
from jax.experimental.pallas import tpu as pltpu
from jax.experimental.pallas import tpu_sc as plsc

NC = 2
NS = 16
NW = NC * NS

SP = 24


@functools.lru_cache(maxsize=None)
def _make_emb(N: int, S: int, D: int):
    npw = N // NW
    assert npw % 2 == 0
    mesh = plsc.VectorSubcoreMesh(core_axis_name="c", subcore_axis_name="s")

    @functools.partial(
        pl.kernel,
        mesh=mesh,
        out_type=jax.ShapeDtypeStruct((N, SP, D), jnp.float32),
        scratch_types=[
            pltpu.VMEM((npw * SP,), jnp.int32),
            pltpu.VMEM((SP, D), jnp.float32),
            pltpu.VMEM((SP, D), jnp.float32),
            pltpu.SemaphoreType.DMA,
            pltpu.SemaphoreType.DMA,
            pltpu.SemaphoreType.DMA,
            pltpu.SemaphoreType.DMA,
        ],
    )
    def emb(table_hbm, idx_hbm, out_hbm, idx_v, b0, b1, g0, g1, s0, s1):
        wid = lax.axis_index("s") * NC + lax.axis_index("c")
        base = wid * npw
        bufs, gsem, ssem = (b0, b1), (g0, g1), (s0, s1)
        pltpu.sync_copy(idx_hbm.at[pl.ds(base * SP, npw * SP)], idx_v)

        def idx(c):
            return idx_v.at[pl.ds(c * SP, SP)]

        def g_start(c, b):
            pltpu.async_copy(table_hbm.at[idx(c)], bufs[b], gsem[b])

        def g_wait(c, b):
            pltpu.make_async_copy(table_hbm.at[idx(c)], bufs[b],
                                  gsem[b]).wait()

        g_start(0, 0)
        g_start(1, 1)

        def body(p, carry):
            for h in range(2):
                c = 2 * p + h
                b = h
                g_wait(c, b)
                pltpu.async_copy(bufs[b], out_hbm.at[base + c], ssem[b])
                pltpu.make_async_copy(bufs[b], out_hbm.at[base + c],
                                      ssem[b]).wait()

                @pl.when(p < npw // 2 - 1)
                def _():
                    g_start(c + 2, b)

            return carry

        lax.fori_loop(0, npw // 2, body, 0)

    return emb


def kernel(x, table):
    N, S = x.shape
    D = table.shape[1]
    xi = x.astype(jnp.int32)
    xe = jnp.concatenate([xi, xi[:, : SP - S]], axis=1).reshape(-1)
    big = _make_emb(N, S, D)(table, xe)
    return big[:, :S, :]
